# Initial kernel scaffold; baseline (speedup 1.0000x reference)
import jax
import jax.numpy as jnp
from jax.experimental import pallas as pl


def kernel(x, edge_index, W_msg, Wr1, Wr2, Wr3, Wrout, Wa, Wb, Wc):
    raise NotImplementedError("write your pallas kernel here")


# speedup vs baseline: 1.2859x; 1.2859x over previous
import functools

import jax
import jax.numpy as jnp
from jax import lax
from jax.experimental import pallas as pl
from jax.experimental.pallas import tpu as pltpu
from jax.experimental.pallas import tpu_sc as plsc

N = 10000
E = 160000
D = 128
DH = 64
NRB = 4
SH_DIM = 9
MCAT = 137
STEPS = 3
X_MAX = 4.0

MP = 144
NC, NS = 2, 16
NW = NC * NS
CHUNK = 128
EPW = 5120
NCH = EPW // CHUNK
EP = NW * EPW
EPS = EP // NS
NCHS = EPS // CHUNK
NPAD = 10240
BE = 1024
BN = 1000


def _sc_gather(h, snd3d, rcv3d):
    mesh = plsc.VectorSubcoreMesh(core_axis_name="c", subcore_axis_name="s")

    @functools.partial(
        pl.kernel,
        mesh=mesh,
        out_type=[
            jax.ShapeDtypeStruct((EP, D), jnp.float32),
            jax.ShapeDtypeStruct((EP, D), jnp.float32),
        ],
        scratch_types=[
            pltpu.VMEM((NCH, CHUNK), jnp.int32),
            pltpu.VMEM((NCH, CHUNK), jnp.int32),
            pltpu.VMEM((CHUNK, D), jnp.float32),
            pltpu.VMEM((CHUNK, D), jnp.float32),
            pltpu.SemaphoreType.DMA,
            pltpu.SemaphoreType.DMA,
        ],
    )
    def gather_kernel(h_hbm, snd_hbm, rcv_hbm, xi_hbm, pj_hbm,
                      sidx, ridx, xiv, pjv, sem1, sem2):
        c = lax.axis_index("c")
        s = lax.axis_index("s")
        w = s * NC + c
        base = w * EPW
        pltpu.sync_copy(snd_hbm.at[w], sidx)
        pltpu.sync_copy(rcv_hbm.at[w], ridx)

        def body(j, carry):
            cp1 = pltpu.async_copy(h_hbm.at[sidx.at[j]], xiv, sem1)
            cp2 = pltpu.async_copy(h_hbm.at[ridx.at[j]], pjv, sem2)
            cp1.wait()
            cp2.wait()
            off = pl.multiple_of(base + j * CHUNK, CHUNK)
            pltpu.sync_copy(xiv, xi_hbm.at[pl.ds(off, CHUNK)])
            pltpu.sync_copy(pjv, pj_hbm.at[pl.ds(off, CHUNK)])
            return carry

        lax.fori_loop(0, NCH, body, 0)

    return gather_kernel(h, snd3d, rcv3d)


def _sc_scatter(mij, rcv3d):
    mesh = plsc.VectorSubcoreMesh(core_axis_name="c", subcore_axis_name="s")
    rows_per_sub = NPAD // NS
    zch = rows_per_sub // CHUNK

    @functools.partial(
        pl.kernel,
        mesh=mesh,
        out_type=jax.ShapeDtypeStruct((NC, NPAD, D), jnp.float32),
        scratch_types=[
            pltpu.VMEM((NCHS, CHUNK), jnp.int32),
            pltpu.VMEM((CHUNK, D), jnp.float32),
            pltpu.VMEM_SHARED((NPAD, D), jnp.float32),
        ],
    )
    def scatter_kernel(mij_hbm, rcv_hbm, out_hbm, idxv, chunk_v, acc):
        c = lax.axis_index("c")
        s = lax.axis_index("s")
        pltpu.sync_copy(mij_hbm.at[c, pl.ds(E, CHUNK)], chunk_v)
        for k in range(zch):
            off = pl.multiple_of((s * zch + k) * CHUNK, CHUNK)
            pltpu.sync_copy(chunk_v, acc.at[pl.ds(off, CHUNK)])
        plsc.subcore_barrier()

        pltpu.sync_copy(rcv_hbm.at[s], idxv)

        def body(j, carry):
            off = pl.multiple_of(s * EPS + j * CHUNK, CHUNK)
            pltpu.sync_copy(mij_hbm.at[c, pl.ds(off, CHUNK)], chunk_v)
            pltpu.sync_copy(chunk_v, acc.at[idxv.at[j]], add=True)
            return carry

        lax.fori_loop(0, NCHS, body, 0)
        plsc.subcore_barrier()

        roff = pl.multiple_of(s * rows_per_sub, CHUNK)
        pltpu.sync_copy(acc.at[pl.ds(roff, rows_per_sub)],
                        out_hbm.at[c, pl.ds(roff, rows_per_sub)])

    return scatter_kernel(mij, rcv3d)


def _edge_body(xi_ref, pj_ref, wm_ref, wr1_ref, wr2_ref, wr3_ref, wro_ref,
               wa_ref, out_ref):
    i = pl.program_id(0)
    f32 = jnp.float32
    xi = xi_ref[...]
    px = xi[:, 0:1] - pj_ref[:, 0:1]
    py = xi[:, 1:2] - pj_ref[:, 1:2]
    pz = xi[:, 2:3] - pj_ref[:, 2:3]
    d2 = px * px + py * py + pz * pz
    d = jnp.sqrt(d2 + 1e-12)
    inv_d = 1.0 / d
    xh, yh, zh = px * inv_d, py * inv_d, pz * inv_d
    s3 = jnp.sqrt(3.0)
    s5 = jnp.sqrt(5.0)
    s15 = jnp.sqrt(15.0)
    comps = (
        jnp.ones_like(xh),
        s3 * xh, s3 * yh, s3 * zh,
        s15 * xh * yh, s15 * yh * zh,
        (s5 / 2.0) * (3.0 * zh * zh - 1.0),
        s15 * xh * zh,
        (s15 / 2.0) * (xh * xh - yh * yh),
    )
    be = xi.shape[0]
    lane16 = lax.broadcasted_iota(jnp.int32, (be, 16), 1)
    sh16 = jnp.zeros((be, 16), f32)
    for k, v in enumerate(comps):
        sh16 = jnp.where(lane16 == k, v, sh16)
    lane_mp = lax.broadcasted_iota(jnp.int32, (be, MP), 1) % SH_DIM
    a_tile = jnp.zeros((be, MP), f32)
    for k, v in enumerate(comps):
        a_tile = jnp.where(lane_mp == k, v, a_tile)

    m = jnp.dot(xi, wm_ref[...], preferred_element_type=f32)
    m_cat = jnp.concatenate([m, sh16], axis=1)
    tp = m_cat * a_tile

    nrow = lax.broadcasted_iota(jnp.int32, (be, 8), 1).astype(f32) + 1.0
    r_basis = (jnp.sqrt(2.0 / X_MAX) * jnp.sin(nrow * (jnp.pi / X_MAX) * d)
               * inv_d)
    h1 = jax.nn.gelu(jnp.dot(r_basis, wr1_ref[...], preferred_element_type=f32))
    h2 = jax.nn.gelu(jnp.dot(h1, wr2_ref[...], preferred_element_type=f32))
    h3 = jax.nn.gelu(jnp.dot(h2, wr3_ref[...], preferred_element_type=f32))
    w_r = jnp.dot(h3, wro_ref[...], preferred_element_type=f32)

    mij = w_r * tp
    ga = jnp.dot(mij, wa_ref[...], preferred_element_type=f32)
    row = i * BE + lax.broadcasted_iota(jnp.int32, (be, 1), 0)
    ga = jnp.where(row < E, ga, 0.0)
    out_ref[0] = ga[:, :D]
    out_ref[1] = ga[:, D:]


def _tc_edge(xi, pj, wm, wr1p, wr2, wr3, wrop, wap):
    grid = (EP // BE,)
    return pl.pallas_call(
        _edge_body,
        grid=grid,
        in_specs=[
            pl.BlockSpec((BE, D), lambda i: (i, 0)),
            pl.BlockSpec((BE, D), lambda i: (i, 0)),
            pl.BlockSpec((D, D), lambda i: (0, 0)),
            pl.BlockSpec((8, DH), lambda i: (0, 0)),
            pl.BlockSpec((DH, DH), lambda i: (0, 0)),
            pl.BlockSpec((DH, DH), lambda i: (0, 0)),
            pl.BlockSpec((DH, MP), lambda i: (0, 0)),
            pl.BlockSpec((MP, 2 * D), lambda i: (0, 0)),
        ],
        out_specs=pl.BlockSpec((NC, BE, D), lambda i: (0, i, 0)),
        out_shape=jax.ShapeDtypeStruct((NC, EP, D), jnp.float32),
        compiler_params=pltpu.CompilerParams(
            dimension_semantics=("arbitrary",),
        ),
    )(xi, pj, wm, wr1p, wr2, wr3, wrop, wap)


def _node_body(acc_ref, h_ref, wb_ref, wc_ref, out_ref):
    f32 = jnp.float32
    gb = jnp.dot(h_ref[...], wb_ref[...], preferred_element_type=f32)
    gates = acc_ref[0] + gb[:, :D]
    feats = acc_ref[1] + gb[:, D:]
    out_ref[...] = jnp.dot(feats * jax.nn.sigmoid(gates), wc_ref[...],
                           preferred_element_type=f32)


def _tc_node(acc, h, wb, wc):
    grid = (N // BN,)
    return pl.pallas_call(
        _node_body,
        grid=grid,
        in_specs=[
            pl.BlockSpec((NC, BN, D), lambda i: (0, i, 0)),
            pl.BlockSpec((BN, D), lambda i: (i, 0)),
            pl.BlockSpec((D, 2 * D), lambda i: (0, 0)),
            pl.BlockSpec((D, D), lambda i: (0, 0)),
        ],
        out_specs=pl.BlockSpec((BN, D), lambda i: (i, 0)),
        out_shape=jax.ShapeDtypeStruct((N, D), jnp.float32),
        compiler_params=pltpu.CompilerParams(
            dimension_semantics=("arbitrary",),
        ),
    )(acc, h, wb, wc)


def kernel(x, edge_index, W_msg, Wr1, Wr2, Wr3, Wrout, Wa, Wb, Wc):
    snd = edge_index[0].astype(jnp.int32)
    rcv = edge_index[1].astype(jnp.int32)
    pad = jnp.zeros((EP - E,), jnp.int32)
    snd_p = jnp.concatenate([snd, pad])
    rcv_p = jnp.concatenate([rcv, pad])
    snd3d = snd_p.reshape(NW, NCH, CHUNK)
    rcv3d = rcv_p.reshape(NW, NCH, CHUNK)
    rcv3s = rcv_p.reshape(NS, NCHS, CHUNK)
    inv_sqrt_e = 1.0 / jnp.sqrt(float(E))

    h = x
    for t in range(STEPS):
        xi, pj = _sc_gather(h, snd3d, rcv3d)
        wr1p = jnp.zeros((8, DH), jnp.float32).at[:NRB].set(Wr1[t])
        wrop = jnp.pad(Wrout[t], ((0, 0), (0, MP - MCAT)))
        wap = jnp.pad(Wa[t] * inv_sqrt_e, ((0, MP - MCAT), (0, 0)))
        mij = _tc_edge(xi, pj, W_msg[t], wr1p, Wr2[t], Wr3[t], wrop, wap)
        acc = _sc_scatter(mij, rcv3s)
        h = _tc_node(acc, h, Wb[t], Wc[t])
    return h


# speedup vs baseline: 2.3543x; 1.8308x over previous
import functools

import jax
import jax.numpy as jnp
from jax import lax
from jax.experimental import pallas as pl
from jax.experimental.pallas import tpu as pltpu
from jax.experimental.pallas import tpu_sc as plsc

N = 10000
E = 160000
D = 128
DH = 64
NRB = 4
SH_DIM = 9
MCAT = 137
STEPS = 3
X_MAX = 4.0

MP = 144
NC, NS = 2, 16
NW = NC * NS
CHUNK = 128
EPW = 5120
NCH = EPW // CHUNK
EP = NW * EPW
EPS = EP // NS
CHS = 64
NCHS = EPS // CHS
NPAD = 10240
BE = 1024
BN = 1000
NBUF = 3


def _sc_gather(h, snd3d, rcv3d):
    mesh = plsc.VectorSubcoreMesh(core_axis_name="c", subcore_axis_name="s")

    @functools.partial(
        pl.kernel,
        mesh=mesh,
        out_type=[
            jax.ShapeDtypeStruct((EP, D), jnp.float32),
            jax.ShapeDtypeStruct((EP, D), jnp.float32),
        ],
        scratch_types=[
            pltpu.VMEM((NCH, CHUNK), jnp.int32),
            pltpu.VMEM((NCH, CHUNK), jnp.int32),
            pltpu.VMEM((NBUF, CHUNK, D), jnp.float32),
            pltpu.VMEM((NBUF, CHUNK, D), jnp.float32),
            pltpu.SemaphoreType.DMA((NBUF,)),
            pltpu.SemaphoreType.DMA((NBUF,)),
            pltpu.SemaphoreType.DMA((NBUF,)),
            pltpu.SemaphoreType.DMA((NBUF,)),
        ],
    )
    def gather_kernel(h_hbm, snd_hbm, rcv_hbm, xi_hbm, pj_hbm,
                      sidx, ridx, xiv, pjv, sgx, sgp, swx, swp):
        c = lax.axis_index("c")
        s = lax.axis_index("s")
        w = s * NC + c
        base = w * EPW
        pltpu.sync_copy(snd_hbm.at[w], sidx)
        pltpu.sync_copy(rcv_hbm.at[w], ridx)

        def issue_gather(j, b):
            pltpu.async_copy(h_hbm.at[sidx.at[j]], xiv.at[b], sgx.at[b])
            pltpu.async_copy(h_hbm.at[ridx.at[j]], pjv.at[b], sgp.at[b])

        def wait_gather(j, b):
            pltpu.make_async_copy(h_hbm.at[sidx.at[j]], xiv.at[b],
                                  sgx.at[b]).wait()
            pltpu.make_async_copy(h_hbm.at[ridx.at[j]], pjv.at[b],
                                  sgp.at[b]).wait()

        def issue_write(j, b):
            off = pl.multiple_of(base + j * CHUNK, CHUNK)
            pltpu.async_copy(xiv.at[b], xi_hbm.at[pl.ds(off, CHUNK)],
                             swx.at[b])
            pltpu.async_copy(pjv.at[b], pj_hbm.at[pl.ds(off, CHUNK)],
                             swp.at[b])

        def wait_write(j, b):
            off = pl.multiple_of(base + j * CHUNK, CHUNK)
            pltpu.make_async_copy(xiv.at[b], xi_hbm.at[pl.ds(off, CHUNK)],
                                  swx.at[b]).wait()
            pltpu.make_async_copy(pjv.at[b], pj_hbm.at[pl.ds(off, CHUNK)],
                                  swp.at[b]).wait()

        issue_gather(0, 0)
        issue_gather(1, 1)

        def body(j, carry):
            b = lax.rem(j, NBUF)
            b2 = lax.rem(j + 2, NBUF)

            @pl.when(j >= 1)
            def _():
                wait_write(j - 1, lax.rem(j - 1, NBUF))

            @pl.when(j + 2 < NCH)
            def _():
                issue_gather(j + 2, b2)

            wait_gather(j, b)
            issue_write(j, b)
            return carry

        lax.fori_loop(0, NCH, body, 0)
        wait_write(NCH - 1, lax.rem(NCH - 1, NBUF))

    return gather_kernel(h, snd3d, rcv3d)


def _sc_scatter(mij, rcv3d):
    mesh = plsc.VectorSubcoreMesh(core_axis_name="c", subcore_axis_name="s")
    rows_per_sub = NPAD // NS
    zch = rows_per_sub // CHS

    @functools.partial(
        pl.kernel,
        mesh=mesh,
        out_type=jax.ShapeDtypeStruct((NC, NPAD, D), jnp.float32),
        scratch_types=[
            pltpu.VMEM((NCHS, CHS), jnp.int32),
            pltpu.VMEM((NBUF, CHS, D), jnp.float32),
            pltpu.VMEM_SHARED((NPAD, D), jnp.float32),
            pltpu.SemaphoreType.DMA((NBUF,)),
            pltpu.SemaphoreType.DMA((NBUF,)),
        ],
    )
    def scatter_kernel(mij_hbm, rcv_hbm, out_hbm, idxv, bufs, acc, srd, sad):
        c = lax.axis_index("c")
        s = lax.axis_index("s")
        pltpu.sync_copy(mij_hbm.at[c, pl.ds(E, CHS)], bufs.at[0])
        for k in range(zch):
            off = pl.multiple_of((s * zch + k) * CHS, CHS)
            pltpu.sync_copy(bufs.at[0], acc.at[pl.ds(off, CHS)])
        plsc.subcore_barrier()

        pltpu.sync_copy(rcv_hbm.at[s], idxv)

        def issue_read(j, b):
            off = pl.multiple_of(s * EPS + j * CHS, CHS)
            pltpu.async_copy(mij_hbm.at[c, pl.ds(off, CHS)], bufs.at[b],
                             srd.at[b])

        def wait_read(j, b):
            off = pl.multiple_of(s * EPS + j * CHS, CHS)
            pltpu.make_async_copy(mij_hbm.at[c, pl.ds(off, CHS)],
                                  bufs.at[b], srd.at[b]).wait()

        def issue_add(j, b):
            pltpu.async_copy(bufs.at[b], acc.at[idxv.at[j]], sad.at[b],
                             add=True)

        def wait_add(j, b):
            pltpu.make_async_copy(bufs.at[b], acc.at[idxv.at[j]],
                                  sad.at[b]).wait()

        issue_read(0, 0)
        issue_read(1, 1)

        def body(j, carry):
            b = lax.rem(j, NBUF)
            b2 = lax.rem(j + 2, NBUF)

            @pl.when(j >= 1)
            def _():
                wait_add(j - 1, lax.rem(j - 1, NBUF))

            @pl.when(j + 2 < NCHS)
            def _():
                issue_read(j + 2, b2)

            wait_read(j, b)
            issue_add(j, b)
            return carry

        lax.fori_loop(0, NCHS, body, 0)
        wait_add(NCHS - 1, lax.rem(NCHS - 1, NBUF))
        plsc.subcore_barrier()

        roff = pl.multiple_of(s * rows_per_sub, CHS)
        pltpu.sync_copy(acc.at[pl.ds(roff, rows_per_sub)],
                        out_hbm.at[c, pl.ds(roff, rows_per_sub)])

    return scatter_kernel(mij, rcv3d)


def _edge_body(xi_ref, pj_ref, wm_ref, wr1_ref, wr2_ref, wr3_ref, wro_ref,
               wa_ref, out_ref):
    i = pl.program_id(0)
    f32 = jnp.float32
    xi = xi_ref[...]
    be = xi.shape[0]
    rsub = xi[:, 0:8] - pj_ref[:, 0:8]
    r_t = jnp.transpose(rsub)
    px, py, pz = r_t[0], r_t[1], r_t[2]
    d2 = px * px + py * py + pz * pz
    d = jnp.sqrt(d2 + 1e-12)
    inv_d = 1.0 / d
    xh, yh, zh = px * inv_d, py * inv_d, pz * inv_d
    s3 = jnp.sqrt(3.0)
    s5 = jnp.sqrt(5.0)
    s15 = jnp.sqrt(15.0)
    comps = (
        jnp.ones_like(xh),
        s3 * xh, s3 * yh, s3 * zh,
        s15 * xh * yh, s15 * yh * zh,
        (s5 / 2.0) * (3.0 * zh * zh - 1.0),
        s15 * xh * zh,
        (s15 / 2.0) * (xh * xh - yh * yh),
    )
    zero1 = jnp.zeros((be,), f32)
    sh_l = jnp.stack(list(comps) + [zero1] * 7, axis=0)

    dims = (((0,), (0,)), ((), ()))
    row16 = lax.broadcasted_iota(jnp.int32, (16, 16), 0)
    col16 = lax.broadcasted_iota(jnp.int32, (16, 16), 1)
    eye16 = (row16 == col16).astype(f32)
    sh16 = lax.dot_general(sh_l, eye16, dims,
                           preferred_element_type=f32)
    rowt = lax.broadcasted_iota(jnp.int32, (16, MP), 0)
    colt = lax.broadcasted_iota(jnp.int32, (16, MP), 1) % SH_DIM
    tmap = (rowt == colt).astype(f32)
    a_tile = lax.dot_general(sh_l, tmap, dims,
                             preferred_element_type=f32)

    m = jnp.dot(xi, wm_ref[...], preferred_element_type=f32)
    m_cat = jnp.concatenate([m, sh16], axis=1)
    tp = m_cat * a_tile

    n8 = (lax.broadcasted_iota(jnp.int32, (8, be), 0) + 1).astype(f32)
    args = n8 * ((jnp.pi / X_MAX) * d)[None, :]
    r_l = jnp.sqrt(2.0 / X_MAX) * jnp.sin(args) * inv_d[None, :]
    h1 = jax.nn.gelu(lax.dot_general(r_l, wr1_ref[...], dims,
                                     preferred_element_type=f32))
    h2 = jax.nn.gelu(jnp.dot(h1, wr2_ref[...], preferred_element_type=f32))
    h3 = jax.nn.gelu(jnp.dot(h2, wr3_ref[...], preferred_element_type=f32))
    w_r = jnp.dot(h3, wro_ref[...], preferred_element_type=f32)

    mij = w_r * tp
    ga = jnp.dot(mij, wa_ref[...], preferred_element_type=f32)
    row = i * BE + lax.broadcasted_iota(jnp.int32, (be, 1), 0)
    ga = jnp.where(row < E, ga, 0.0)
    out_ref[0] = ga[:, :D]
    out_ref[1] = ga[:, D:]


def _tc_edge(xi, pj, wm, wr1p, wr2, wr3, wrop, wap):
    grid = (EP // BE,)
    return pl.pallas_call(
        _edge_body,
        grid=grid,
        in_specs=[
            pl.BlockSpec((BE, D), lambda i: (i, 0)),
            pl.BlockSpec((BE, D), lambda i: (i, 0)),
            pl.BlockSpec((D, D), lambda i: (0, 0)),
            pl.BlockSpec((8, DH), lambda i: (0, 0)),
            pl.BlockSpec((DH, DH), lambda i: (0, 0)),
            pl.BlockSpec((DH, DH), lambda i: (0, 0)),
            pl.BlockSpec((DH, MP), lambda i: (0, 0)),
            pl.BlockSpec((MP, 2 * D), lambda i: (0, 0)),
        ],
        out_specs=pl.BlockSpec((NC, BE, D), lambda i: (0, i, 0)),
        out_shape=jax.ShapeDtypeStruct((NC, EP, D), jnp.float32),
        compiler_params=pltpu.CompilerParams(
            dimension_semantics=("arbitrary",),
        ),
    )(xi, pj, wm, wr1p, wr2, wr3, wrop, wap)


def _node_body(acc_ref, h_ref, wb_ref, wc_ref, out_ref):
    f32 = jnp.float32
    gb = jnp.dot(h_ref[...], wb_ref[...], preferred_element_type=f32)
    gates = acc_ref[0] + gb[:, :D]
    feats = acc_ref[1] + gb[:, D:]
    out_ref[...] = jnp.dot(feats * jax.nn.sigmoid(gates), wc_ref[...],
                           preferred_element_type=f32)


def _tc_node(acc, h, wb, wc):
    grid = (N // BN,)
    return pl.pallas_call(
        _node_body,
        grid=grid,
        in_specs=[
            pl.BlockSpec((NC, BN, D), lambda i: (0, i, 0)),
            pl.BlockSpec((BN, D), lambda i: (i, 0)),
            pl.BlockSpec((D, 2 * D), lambda i: (0, 0)),
            pl.BlockSpec((D, D), lambda i: (0, 0)),
        ],
        out_specs=pl.BlockSpec((BN, D), lambda i: (i, 0)),
        out_shape=jax.ShapeDtypeStruct((N, D), jnp.float32),
        compiler_params=pltpu.CompilerParams(
            dimension_semantics=("arbitrary",),
        ),
    )(acc, h, wb, wc)


def kernel(x, edge_index, W_msg, Wr1, Wr2, Wr3, Wrout, Wa, Wb, Wc):
    snd = edge_index[0].astype(jnp.int32)
    rcv = edge_index[1].astype(jnp.int32)
    pad = jnp.zeros((EP - E,), jnp.int32)
    snd_p = jnp.concatenate([snd, pad])
    rcv_p = jnp.concatenate([rcv, pad])
    snd3d = snd_p.reshape(NW, NCH, CHUNK)
    rcv3d = rcv_p.reshape(NW, NCH, CHUNK)
    rcv3s = rcv_p.reshape(NS, NCHS, CHS)
    inv_sqrt_e = 1.0 / jnp.sqrt(float(E))

    h = x
    for t in range(STEPS):
        xi, pj = _sc_gather(h, snd3d, rcv3d)
        wr1p = jnp.zeros((8, DH), jnp.float32).at[:NRB].set(Wr1[t])
        wrop = jnp.pad(Wrout[t], ((0, 0), (0, MP - MCAT)))
        wap = jnp.pad(Wa[t] * inv_sqrt_e, ((0, MP - MCAT), (0, 0)))
        mij = _tc_edge(xi, pj, W_msg[t], wr1p, Wr2[t], Wr3[t], wrop, wap)
        acc = _sc_scatter(mij, rcv3s)
        h = _tc_node(acc, h, Wb[t], Wc[t])
    return h
